# Initial kernel scaffold; baseline (speedup 1.0000x reference)
import functools

import jax
import jax.numpy as jnp
from jax import lax
from jax.experimental import pallas as pl
from jax.experimental.pallas import tpu as pltpu
from jax.experimental.pallas import tpu_sc as plsc

N = 10000
E = 320000
H = 128
NUM_LAYERS_ = 4

NC = 2
NS = 16
NW = NC * NS
EPW = E // NW
CH = 80
NCHUNK = EPW // CH
NP = 10240
RPT = NP // NS

_mesh = plsc.VectorSubcoreMesh(core_axis_name="c", subcore_axis_name="s")


@functools.partial(
    pl.kernel,
    out_type=(
        jax.ShapeDtypeStruct((NW, NCHUNK, CH), jnp.int32),
        jax.ShapeDtypeStruct((NC, NP), jnp.float32),
    ),
    mesh=_mesh,
    scratch_types=[
        pltpu.VMEM((NCHUNK, CH), jnp.int32),
        pltpu.VMEM((NCHUNK, CH), jnp.int32),
        pltpu.VMEM((CH,), jnp.float32),
        pltpu.VMEM((RPT,), jnp.float32),
        pltpu.VMEM_SHARED((NP,), jnp.float32),
    ],
)
def _sc_deg(row3, col3, col2_out, deg_out, row_v, col_v, ones_v, zb_v, acc):
    c = lax.axis_index("c")
    s = lax.axis_index("s")
    w = c * NS + s
    pltpu.sync_copy(row3.at[w], row_v)
    pltpu.sync_copy(col3.at[w], col_v)

    for k in range(CH // 16):
        ones_v[pl.ds(k * 16, 16)] = jnp.full((16,), 1.0, jnp.float32)

    def zfill(i, _):
        zb_v[pl.ds(i * 16, 16)] = jnp.zeros((16,), jnp.float32)
        return 0

    lax.fori_loop(0, RPT // 16, zfill, 0, unroll=8)
    pltpu.sync_copy(zb_v, acc.at[pl.ds(s * RPT, RPT)])

    def rewrite(j, _):
        for k in range(CH // 16):
            r16 = row_v[j, pl.ds(k * 16, 16)]
            c16 = col_v[j, pl.ds(k * 16, 16)]
            col_v[j, pl.ds(k * 16, 16)] = jnp.where(r16 == c16, N, c16)
        return 0

    lax.fori_loop(0, NCHUNK, rewrite, 0)
    pltpu.sync_copy(col_v, col2_out.at[w])
    plsc.subcore_barrier()

    def body(j, _):
        pltpu.sync_copy(ones_v, acc.at[col_v.at[j]], add=True)
        return 0

    lax.fori_loop(0, NCHUNK, body, 0)
    plsc.subcore_barrier()
    pltpu.sync_copy(acc.at[pl.ds(s * RPT, RPT)], deg_out.at[c, pl.ds(s * RPT, RPT)])


@functools.partial(
    pl.kernel,
    out_type=jax.ShapeDtypeStruct((NC, NP, H), jnp.float32),
    mesh=_mesh,
    scratch_types=[
        pltpu.VMEM((NCHUNK, CH), jnp.int32),
        pltpu.VMEM((NCHUNK, CH), jnp.int32),
        pltpu.VMEM((CH, H), jnp.float32),
        pltpu.VMEM((CH, H), jnp.float32),
        pltpu.SemaphoreType.DMA,
        pltpu.SemaphoreType.DMA,
        pltpu.VMEM_SHARED((NP, H), jnp.float32),
    ],
)
def _sc_agg(hp, row3, col23, zeros2, u_out, row_v, col_v, g0, g1, sem0, sem1, acc):
    c = lax.axis_index("c")
    s = lax.axis_index("s")
    w = c * NS + s
    pltpu.sync_copy(row3.at[w], row_v)
    pltpu.sync_copy(col23.at[w], col_v)
    pltpu.sync_copy(zeros2.at[pl.ds(s * RPT, RPT)], acc.at[pl.ds(s * RPT, RPT)])
    plsc.subcore_barrier()

    pltpu.async_copy(hp.at[row_v.at[0]], g0, sem0)

    def body(j, _):
        even = j % 2 == 0

        @pl.when(jnp.logical_and(even, j + 1 < NCHUNK))
        def _():
            pltpu.async_copy(hp.at[row_v.at[j + 1]], g1, sem1)

        @pl.when(jnp.logical_and(jnp.logical_not(even), j + 1 < NCHUNK))
        def _():
            pltpu.async_copy(hp.at[row_v.at[j + 1]], g0, sem0)

        @pl.when(even)
        def _():
            pltpu.make_async_copy(hp.at[row_v.at[j]], g0, sem0).wait()
            pltpu.sync_copy(g0, acc.at[col_v.at[j]], add=True)

        @pl.when(jnp.logical_not(even))
        def _():
            pltpu.make_async_copy(hp.at[row_v.at[j]], g1, sem1).wait()
            pltpu.sync_copy(g1, acc.at[col_v.at[j]], add=True)

        return 0

    lax.fori_loop(0, NCHUNK, body, 0)
    plsc.subcore_barrier()
    pltpu.sync_copy(acc.at[pl.ds(s * RPT, RPT)], u_out.at[c, pl.ds(s * RPT, RPT)])


def _tc_prep_body(deg_ref, wa_ref, wpr_ref, dis_ref, wp_ref, at_ref):
    r = lax.broadcasted_iota(jnp.int32, (H, H), 0)
    col = lax.broadcasted_iota(jnp.int32, (H, H), 1)
    eye = (r == col).astype(jnp.float32)
    w0 = jnp.where(col > r, wpr_ref[:, :H], 0.0)
    w0t = lax.dot_general(w0, eye, (((0,), (0,)), ((), ())),
                          preferred_element_type=jnp.float32)
    w0s = w0 + w0t
    q = wpr_ref[:, H:H + 1]
    rr = wpr_ref[:, H + 1:H + 2]
    d = q * jnp.sum(jnp.abs(w0s), axis=1, keepdims=True) + rr
    wp_ref[...] = w0s + eye * d
    wa = wa_ref[...]
    wat = lax.dot_general(wa, eye, (((0,), (0,)), ((), ())),
                          preferred_element_type=jnp.float32)
    at_ref[...] = wat - wa
    deg = deg_ref[0] + deg_ref[1]
    dis_ref[...] = jnp.where(deg > 0, lax.rsqrt(jnp.maximum(deg, 1e-30)), 0.0)


_tc_prep = pl.pallas_call(
    _tc_prep_body,
    out_shape=(
        jax.ShapeDtypeStruct((NP // 128, 128), jnp.float32),
        jax.ShapeDtypeStruct((H, H), jnp.float32),
        jax.ShapeDtypeStruct((H, H), jnp.float32),
    ),
)

_RB = 2000


def _tc_proj_body(x_ref, wi_ref, b_ref, dis_ref, h_ref, hp_ref):
    hb = lax.dot_general(x_ref[...], wi_ref[...], (((1,), (1,)), ((), ())),
                         preferred_element_type=jnp.float32)
    hb = jnp.maximum(hb + b_ref[...], 0.0)
    h_ref[...] = hb
    hp_ref[...] = dis_ref[...] * hb


_tc_proj = pl.pallas_call(
    _tc_proj_body,
    grid=(N // _RB,),
    in_specs=[
        pl.BlockSpec((_RB, H), lambda i: (i, 0)),
        pl.BlockSpec((H, H), lambda i: (0, 0)),
        pl.BlockSpec((1, H), lambda i: (0, 0)),
        pl.BlockSpec((_RB, 1), lambda i: (i, 0)),
    ],
    out_specs=(
        pl.BlockSpec((_RB, H), lambda i: (i, 0)),
        pl.BlockSpec((_RB, H), lambda i: (i, 0)),
    ),
    out_shape=(
        jax.ShapeDtypeStruct((N, H), jnp.float32),
        jax.ShapeDtypeStruct((N, H), jnp.float32),
    ),
)


def _tc_update_body(h_ref, u0_ref, u1_ref, dis_ref, wp_ref, at_ref,
                    hn_ref, hp_ref):
    usum = u0_ref[...] + u1_ref[...]
    agg = dis_ref[...] * lax.dot_general(
        usum, wp_ref[...], (((1,), (0,)), ((), ())),
        preferred_element_type=jnp.float32)
    t = jnp.maximum(lax.dot_general(
        h_ref[...], at_ref[...], (((1,), (0,)), ((), ())),
        preferred_element_type=jnp.float32), 0.0)
    hn = h_ref[...] + jnp.maximum(jnp.tanh(agg - t), 0.0)
    hn_ref[...] = hn
    hp_ref[...] = dis_ref[...] * hn


_tc_update = pl.pallas_call(
    _tc_update_body,
    grid=(N // _RB,),
    in_specs=[
        pl.BlockSpec((_RB, H), lambda i: (i, 0)),
        pl.BlockSpec((_RB, H), lambda i: (i, 0)),
        pl.BlockSpec((_RB, H), lambda i: (i, 0)),
        pl.BlockSpec((_RB, 1), lambda i: (i, 0)),
        pl.BlockSpec((H, H), lambda i: (0, 0)),
        pl.BlockSpec((H, H), lambda i: (0, 0)),
    ],
    out_specs=(
        pl.BlockSpec((_RB, H), lambda i: (i, 0)),
        pl.BlockSpec((_RB, H), lambda i: (i, 0)),
    ),
    out_shape=(
        jax.ShapeDtypeStruct((N, H), jnp.float32),
        jax.ShapeDtypeStruct((N, H), jnp.float32),
    ),
)


def kernel(x, edge_index, W_in, b_in, W_anti, W_pair):
    row3 = edge_index[0].reshape(NW, NCHUNK, CH)
    col3 = edge_index[1].reshape(NW, NCHUNK, CH)
    zeros2 = jnp.zeros((NP, H), jnp.float32)

    col23, deg2 = _sc_deg(row3, col3)
    dis2d, Wp, A_T = _tc_prep(deg2.reshape(NC, NP // 128, 128), W_anti, W_pair)
    dis = dis2d.reshape(NP)[:N].reshape(N, 1)

    h, hp = _tc_proj(x, W_in, b_in.reshape(1, H), dis)
    for _ in range(NUM_LAYERS_):
        u = _sc_agg(hp, row3, col23, zeros2)
        h, hp = _tc_update(h, u[0, :N], u[1, :N], dis, Wp, A_T)
    return h


# speedup vs baseline: 16.7402x; 16.7402x over previous
import functools

import jax
import jax.numpy as jnp
from jax import lax
from jax.experimental import pallas as pl
from jax.experimental.pallas import tpu as pltpu
from jax.experimental.pallas import tpu_sc as plsc

N = 10000
E = 320000
H = 128
HH = H // 2
NUM_LAYERS_ = 4

NC = 2
NS = 16
NW = NC * NS
CH = 80
EPW = E // NW
NCHUNK = EPW // CH
EPS = E // NS
NCHUNK2 = EPS // CH
NP = 10240
RPT = NP // NS

_mesh = plsc.VectorSubcoreMesh(core_axis_name="c", subcore_axis_name="s")


@functools.partial(
    pl.kernel,
    out_type=(
        jax.ShapeDtypeStruct((NW, NCHUNK, CH), jnp.int32),
        jax.ShapeDtypeStruct((NC, NP), jnp.float32),
    ),
    mesh=_mesh,
    scratch_types=[
        pltpu.VMEM((NCHUNK, CH), jnp.int32),
        pltpu.VMEM((NCHUNK, CH), jnp.int32),
        pltpu.VMEM((CH,), jnp.float32),
        pltpu.VMEM((RPT,), jnp.float32),
        pltpu.VMEM_SHARED((NP,), jnp.float32),
    ],
)
def _sc_deg(row3, col3, col2_out, deg_out, row_v, col_v, ones_v, zb_v, acc):
    c = lax.axis_index("c")
    s = lax.axis_index("s")
    w = c * NS + s
    pltpu.sync_copy(row3.at[w], row_v)
    pltpu.sync_copy(col3.at[w], col_v)

    for k in range(CH // 16):
        ones_v[pl.ds(k * 16, 16)] = jnp.full((16,), 1.0, jnp.float32)

    def zfill(i, _):
        zb_v[pl.ds(i * 16, 16)] = jnp.zeros((16,), jnp.float32)
        return 0

    lax.fori_loop(0, RPT // 16, zfill, 0, unroll=8)
    pltpu.sync_copy(zb_v, acc.at[pl.ds(s * RPT, RPT)])

    def rewrite(j, _):
        for k in range(CH // 16):
            r16 = row_v[j, pl.ds(k * 16, 16)]
            c16 = col_v[j, pl.ds(k * 16, 16)]
            col_v[j, pl.ds(k * 16, 16)] = jnp.where(r16 == c16, N, c16)
        return 0

    lax.fori_loop(0, NCHUNK, rewrite, 0)
    pltpu.sync_copy(col_v, col2_out.at[w])
    plsc.subcore_barrier()

    def body(j, _):
        pltpu.sync_copy(ones_v, acc.at[col_v.at[j]], add=True)
        return 0

    lax.fori_loop(0, NCHUNK, body, 0)
    plsc.subcore_barrier()
    pltpu.sync_copy(acc.at[pl.ds(s * RPT, RPT)], deg_out.at[c, pl.ds(s * RPT, RPT)])


SB = 25
NSTAGE = NCHUNK // SB


@functools.partial(
    pl.kernel,
    out_type=jax.ShapeDtypeStruct((NC, NP, H), jnp.float32),
    mesh=_mesh,
    scratch_types=[
        pltpu.VMEM((SB, CH), jnp.int32),
        pltpu.VMEM((SB, CH), jnp.int32),
        pltpu.VMEM((CH, H), jnp.float32),
        pltpu.VMEM((CH, H), jnp.float32),
        pltpu.SemaphoreType.DMA,
        pltpu.SemaphoreType.DMA,
        pltpu.VMEM_SHARED((NP, H), jnp.float32),
    ],
)
def _sc_agg(hp, row3, col23, zeros2, u_out, row_v, col_v, g0, g1, sem0, sem1, acc):
    c = lax.axis_index("c")
    s = lax.axis_index("s")
    w = c * NS + s
    pltpu.sync_copy(zeros2.at[pl.ds(s * RPT, RPT)], acc.at[pl.ds(s * RPT, RPT)])
    plsc.subcore_barrier()

    def stage(st, _):
        pltpu.sync_copy(row3.at[w, st], row_v)
        pltpu.sync_copy(col23.at[w, st], col_v)

        pltpu.async_copy(hp.at[row_v.at[0]], g0, sem0)

        def body(j, _):
            even = j % 2 == 0

            @pl.when(jnp.logical_and(even, j + 1 < SB))
            def _():
                pltpu.async_copy(hp.at[row_v.at[j + 1]], g1, sem1)

            @pl.when(jnp.logical_and(jnp.logical_not(even), j + 1 < SB))
            def _():
                pltpu.async_copy(hp.at[row_v.at[j + 1]], g0, sem0)

            @pl.when(even)
            def _():
                pltpu.make_async_copy(hp.at[row_v.at[j]], g0, sem0).wait()
                pltpu.sync_copy(g0, acc.at[col_v.at[j]], add=True)

            @pl.when(jnp.logical_not(even))
            def _():
                pltpu.make_async_copy(hp.at[row_v.at[j]], g1, sem1).wait()
                pltpu.sync_copy(g1, acc.at[col_v.at[j]], add=True)

            return 0

        lax.fori_loop(0, SB, body, 0)
        return 0

    lax.fori_loop(0, NSTAGE, stage, 0)
    plsc.subcore_barrier()
    pltpu.sync_copy(acc.at[pl.ds(s * RPT, RPT)], u_out.at[c, pl.ds(s * RPT, RPT)])


def _tc_prep_body(deg_ref, wa_ref, wpr_ref, dis_ref, wp_ref, at_ref):
    r = lax.broadcasted_iota(jnp.int32, (H, H), 0)
    col = lax.broadcasted_iota(jnp.int32, (H, H), 1)
    eye = (r == col).astype(jnp.float32)
    w0 = jnp.where(col > r, wpr_ref[:, :H], 0.0)
    w0t = lax.dot_general(w0, eye, (((0,), (0,)), ((), ())),
                          preferred_element_type=jnp.float32)
    w0s = w0 + w0t
    q = wpr_ref[:, H:H + 1]
    rr = wpr_ref[:, H + 1:H + 2]
    d = q * jnp.sum(jnp.abs(w0s), axis=1, keepdims=True) + rr
    wp_ref[...] = w0s + eye * d
    wa = wa_ref[...]
    wat = lax.dot_general(wa, eye, (((0,), (0,)), ((), ())),
                          preferred_element_type=jnp.float32)
    at_ref[...] = wat - wa
    deg = deg_ref[0] + deg_ref[1]
    dis_ref[...] = jnp.where(deg > 0, lax.rsqrt(jnp.maximum(deg, 1e-30)), 0.0)


_tc_prep = pl.pallas_call(
    _tc_prep_body,
    out_shape=(
        jax.ShapeDtypeStruct((NP // 128, 128), jnp.float32),
        jax.ShapeDtypeStruct((H, H), jnp.float32),
        jax.ShapeDtypeStruct((H, H), jnp.float32),
    ),
)

_RB = 2000


def _tc_proj_body(x_ref, wi_ref, b_ref, dis_ref, h_ref, hp_ref):
    hb = lax.dot_general(x_ref[...], wi_ref[...], (((1,), (1,)), ((), ())),
                         preferred_element_type=jnp.float32)
    hb = jnp.maximum(hb + b_ref[...], 0.0)
    h_ref[...] = hb
    hp_ref[...] = dis_ref[...] * hb


_tc_proj = pl.pallas_call(
    _tc_proj_body,
    grid=(N // _RB,),
    in_specs=[
        pl.BlockSpec((_RB, H), lambda i: (i, 0)),
        pl.BlockSpec((H, H), lambda i: (0, 0)),
        pl.BlockSpec((1, H), lambda i: (0, 0)),
        pl.BlockSpec((_RB, 1), lambda i: (i, 0)),
    ],
    out_specs=(
        pl.BlockSpec((_RB, H), lambda i: (i, 0)),
        pl.BlockSpec((_RB, H), lambda i: (i, 0)),
    ),
    out_shape=(
        jax.ShapeDtypeStruct((N, H), jnp.float32),
        jax.ShapeDtypeStruct((N, H), jnp.float32),
    ),
)


def _tc_update_body(h_ref, u0_ref, u1_ref, dis_ref, wp_ref, at_ref,
                    hn_ref, hp_ref):
    usum = u0_ref[...] + u1_ref[...]
    agg = dis_ref[...] * lax.dot_general(
        usum, wp_ref[...], (((1,), (0,)), ((), ())),
        preferred_element_type=jnp.float32)
    t = jnp.maximum(lax.dot_general(
        h_ref[...], at_ref[...], (((1,), (0,)), ((), ())),
        preferred_element_type=jnp.float32), 0.0)
    hn = h_ref[...] + jnp.maximum(jnp.tanh(agg - t), 0.0)
    hn_ref[...] = hn
    hp_ref[...] = dis_ref[...] * hn


_tc_update = pl.pallas_call(
    _tc_update_body,
    grid=(N // _RB,),
    in_specs=[
        pl.BlockSpec((_RB, H), lambda i: (i, 0)),
        pl.BlockSpec((_RB, H), lambda i: (i, 0)),
        pl.BlockSpec((_RB, H), lambda i: (i, 0)),
        pl.BlockSpec((_RB, 1), lambda i: (i, 0)),
        pl.BlockSpec((H, H), lambda i: (0, 0)),
        pl.BlockSpec((H, H), lambda i: (0, 0)),
    ],
    out_specs=(
        pl.BlockSpec((_RB, H), lambda i: (i, 0)),
        pl.BlockSpec((_RB, H), lambda i: (i, 0)),
    ),
    out_shape=(
        jax.ShapeDtypeStruct((N, H), jnp.float32),
        jax.ShapeDtypeStruct((N, H), jnp.float32),
    ),
)


def kernel(x, edge_index, W_in, b_in, W_anti, W_pair):
    row3 = edge_index[0].reshape(NW, NCHUNK, CH)
    col3 = edge_index[1].reshape(NW, NCHUNK, CH)
    zeros2 = jnp.zeros((NP, H), jnp.float32)

    col23, deg2 = _sc_deg(row3, col3)
    dis2d, Wp, A_T = _tc_prep(deg2.reshape(NC, NP // 128, 128), W_anti, W_pair)
    dis = dis2d.reshape(NP)[:N].reshape(N, 1)

    row4 = row3.reshape(NW, NSTAGE, SB, CH)
    col24 = col23.reshape(NW, NSTAGE, SB, CH)

    h, hp = _tc_proj(x, W_in, b_in.reshape(1, H), dis)
    for _ in range(NUM_LAYERS_):
        u = _sc_agg(hp, row4, col24, zeros2)
        h, hp = _tc_update(h, u[0, :N], u[1, :N], dis, Wp, A_T)
    return h


# speedup vs baseline: 19.0611x; 1.1386x over previous
import functools

import jax
import jax.numpy as jnp
from jax import lax
from jax.experimental import pallas as pl
from jax.experimental.pallas import tpu as pltpu
from jax.experimental.pallas import tpu_sc as plsc

N = 10000
E = 320000
H = 128
HH = H // 2
NUM_LAYERS_ = 4

NC = 2
NS = 16
NW = NC * NS
CH = 80
EPW = E // NW
NCHUNK = EPW // CH
EPS = E // NS
NCHUNK2 = EPS // CH
NP = 10240
RPT = NP // NS

_mesh = plsc.VectorSubcoreMesh(core_axis_name="c", subcore_axis_name="s")


@functools.partial(
    pl.kernel,
    out_type=(
        jax.ShapeDtypeStruct((NW, NCHUNK, CH), jnp.int32),
        jax.ShapeDtypeStruct((NC, NP), jnp.float32),
    ),
    mesh=_mesh,
    scratch_types=[
        pltpu.VMEM((NCHUNK, CH), jnp.int32),
        pltpu.VMEM((NCHUNK, CH), jnp.int32),
        pltpu.VMEM((CH,), jnp.float32),
        pltpu.VMEM((RPT,), jnp.float32),
        pltpu.VMEM_SHARED((NP,), jnp.float32),
    ],
)
def _sc_deg(row3, col3, col2_out, deg_out, row_v, col_v, ones_v, zb_v, acc):
    c = lax.axis_index("c")
    s = lax.axis_index("s")
    w = c * NS + s
    pltpu.sync_copy(row3.at[w], row_v)
    pltpu.sync_copy(col3.at[w], col_v)

    for k in range(CH // 16):
        ones_v[pl.ds(k * 16, 16)] = jnp.full((16,), 1.0, jnp.float32)

    def zfill(i, _):
        zb_v[pl.ds(i * 16, 16)] = jnp.zeros((16,), jnp.float32)
        return 0

    lax.fori_loop(0, RPT // 16, zfill, 0, unroll=8)
    pltpu.sync_copy(zb_v, acc.at[pl.ds(s * RPT, RPT)])

    def rewrite(j, _):
        for k in range(CH // 16):
            r16 = row_v[j, pl.ds(k * 16, 16)]
            c16 = col_v[j, pl.ds(k * 16, 16)]
            col_v[j, pl.ds(k * 16, 16)] = jnp.where(r16 == c16, N, c16)
        return 0

    lax.fori_loop(0, NCHUNK, rewrite, 0)
    pltpu.sync_copy(col_v, col2_out.at[w])
    plsc.subcore_barrier()

    def body(j, _):
        pltpu.sync_copy(ones_v, acc.at[col_v.at[j]], add=True)
        return 0

    lax.fori_loop(0, NCHUNK, body, 0)
    plsc.subcore_barrier()
    pltpu.sync_copy(acc.at[pl.ds(s * RPT, RPT)], deg_out.at[c, pl.ds(s * RPT, RPT)])


SB = 25
NSTAGE = NCHUNK // SB


NR = 3


@functools.partial(
    pl.kernel,
    out_type=jax.ShapeDtypeStruct((NC, NP, H), jnp.float32),
    mesh=_mesh,
    scratch_types=[
        pltpu.VMEM((SB, CH), jnp.int32),
        pltpu.VMEM((SB, CH), jnp.int32),
        pltpu.VMEM((CH, H), jnp.float32),
        pltpu.VMEM((CH, H), jnp.float32),
        pltpu.VMEM((CH, H), jnp.float32),
        pltpu.SemaphoreType.DMA,
        pltpu.SemaphoreType.DMA,
        pltpu.SemaphoreType.DMA,
        pltpu.SemaphoreType.DMA,
        pltpu.SemaphoreType.DMA,
        pltpu.SemaphoreType.DMA,
        pltpu.VMEM_SHARED((NP, H), jnp.float32),
    ],
)
def _sc_agg(hp, row3, col23, zeros2, u_out, row_v, col_v,
            g0, g1, g2, gs0, gs1, gs2, ss0, ss1, ss2, acc):
    c = lax.axis_index("c")
    s = lax.axis_index("s")
    w = c * NS + s
    gbufs = (g0, g1, g2)
    gsems = (gs0, gs1, gs2)
    ssems = (ss0, ss1, ss2)
    pltpu.sync_copy(zeros2.at[pl.ds(s * RPT, RPT)], acc.at[pl.ds(s * RPT, RPT)])
    plsc.subcore_barrier()

    def stage(st, _):
        pltpu.sync_copy(row3.at[w, st], row_v)
        pltpu.sync_copy(col23.at[w, st], col_v)

        for b in range(NR):
            pltpu.async_copy(hp.at[row_v.at[b]], gbufs[b], gsems[b])

        def body(j, _):
            @pl.when(j >= 1)
            def _():
                for b in range(NR):
                    @pl.when((j - 1) % NR == b)
                    def _():
                        pltpu.make_async_copy(
                            gbufs[b], acc.at[col_v.at[j - 1]], ssems[b]).wait()

                        @pl.when(j + NR - 1 < SB)
                        def _():
                            pltpu.async_copy(
                                hp.at[row_v.at[j + NR - 1]], gbufs[b], gsems[b])

            for b in range(NR):
                @pl.when(j % NR == b)
                def _():
                    pltpu.make_async_copy(
                        hp.at[row_v.at[j]], gbufs[b], gsems[b]).wait()
                    pltpu.async_copy(
                        gbufs[b], acc.at[col_v.at[j]], ssems[b], add=True)

            return 0

        lax.fori_loop(0, SB, body, 0)
        for b in range(NR):
            if (SB - 1) % NR == b:
                pltpu.make_async_copy(
                    gbufs[b], acc.at[col_v.at[SB - 1]], ssems[b]).wait()
        return 0

    lax.fori_loop(0, NSTAGE, stage, 0)
    plsc.subcore_barrier()
    pltpu.sync_copy(acc.at[pl.ds(s * RPT, RPT)], u_out.at[c, pl.ds(s * RPT, RPT)])


def _tc_prep_body(deg_ref, wa_ref, wpr_ref, dis_ref, wp_ref, at_ref):
    r = lax.broadcasted_iota(jnp.int32, (H, H), 0)
    col = lax.broadcasted_iota(jnp.int32, (H, H), 1)
    eye = (r == col).astype(jnp.float32)
    w0 = jnp.where(col > r, wpr_ref[:, :H], 0.0)
    w0t = lax.dot_general(w0, eye, (((0,), (0,)), ((), ())),
                          preferred_element_type=jnp.float32)
    w0s = w0 + w0t
    q = wpr_ref[:, H:H + 1]
    rr = wpr_ref[:, H + 1:H + 2]
    d = q * jnp.sum(jnp.abs(w0s), axis=1, keepdims=True) + rr
    wp_ref[...] = w0s + eye * d
    wa = wa_ref[...]
    wat = lax.dot_general(wa, eye, (((0,), (0,)), ((), ())),
                          preferred_element_type=jnp.float32)
    at_ref[...] = wat - wa
    deg = deg_ref[0] + deg_ref[1]
    dis_ref[...] = jnp.where(deg > 0, lax.rsqrt(jnp.maximum(deg, 1e-30)), 0.0)


_tc_prep = pl.pallas_call(
    _tc_prep_body,
    out_shape=(
        jax.ShapeDtypeStruct((NP // 128, 128), jnp.float32),
        jax.ShapeDtypeStruct((H, H), jnp.float32),
        jax.ShapeDtypeStruct((H, H), jnp.float32),
    ),
)

_RB = 2000


def _tc_proj_body(x_ref, wi_ref, b_ref, dis_ref, h_ref, hp_ref):
    hb = lax.dot_general(x_ref[...], wi_ref[...], (((1,), (1,)), ((), ())),
                         preferred_element_type=jnp.float32)
    hb = jnp.maximum(hb + b_ref[...], 0.0)
    h_ref[...] = hb
    hp_ref[...] = dis_ref[...] * hb


_tc_proj = pl.pallas_call(
    _tc_proj_body,
    grid=(N // _RB,),
    in_specs=[
        pl.BlockSpec((_RB, H), lambda i: (i, 0)),
        pl.BlockSpec((H, H), lambda i: (0, 0)),
        pl.BlockSpec((1, H), lambda i: (0, 0)),
        pl.BlockSpec((_RB, 1), lambda i: (i, 0)),
    ],
    out_specs=(
        pl.BlockSpec((_RB, H), lambda i: (i, 0)),
        pl.BlockSpec((_RB, H), lambda i: (i, 0)),
    ),
    out_shape=(
        jax.ShapeDtypeStruct((N, H), jnp.float32),
        jax.ShapeDtypeStruct((N, H), jnp.float32),
    ),
)


def _tc_update_body(h_ref, u0_ref, u1_ref, dis_ref, wp_ref, at_ref,
                    hn_ref, hp_ref):
    usum = u0_ref[...] + u1_ref[...]
    agg = dis_ref[...] * lax.dot_general(
        usum, wp_ref[...], (((1,), (0,)), ((), ())),
        preferred_element_type=jnp.float32)
    t = jnp.maximum(lax.dot_general(
        h_ref[...], at_ref[...], (((1,), (0,)), ((), ())),
        preferred_element_type=jnp.float32), 0.0)
    hn = h_ref[...] + jnp.maximum(jnp.tanh(agg - t), 0.0)
    hn_ref[...] = hn
    hp_ref[...] = dis_ref[...] * hn


_tc_update = pl.pallas_call(
    _tc_update_body,
    grid=(N // _RB,),
    in_specs=[
        pl.BlockSpec((_RB, H), lambda i: (i, 0)),
        pl.BlockSpec((_RB, H), lambda i: (i, 0)),
        pl.BlockSpec((_RB, H), lambda i: (i, 0)),
        pl.BlockSpec((_RB, 1), lambda i: (i, 0)),
        pl.BlockSpec((H, H), lambda i: (0, 0)),
        pl.BlockSpec((H, H), lambda i: (0, 0)),
    ],
    out_specs=(
        pl.BlockSpec((_RB, H), lambda i: (i, 0)),
        pl.BlockSpec((_RB, H), lambda i: (i, 0)),
    ),
    out_shape=(
        jax.ShapeDtypeStruct((N, H), jnp.float32),
        jax.ShapeDtypeStruct((N, H), jnp.float32),
    ),
)


def kernel(x, edge_index, W_in, b_in, W_anti, W_pair):
    row3 = edge_index[0].reshape(NW, NCHUNK, CH)
    col3 = edge_index[1].reshape(NW, NCHUNK, CH)
    zeros2 = jnp.zeros((NP, H), jnp.float32)

    col23, deg2 = _sc_deg(row3, col3)
    dis2d, Wp, A_T = _tc_prep(deg2.reshape(NC, NP // 128, 128), W_anti, W_pair)
    dis = dis2d.reshape(NP)[:N].reshape(N, 1)

    row4 = row3.reshape(NW, NSTAGE, SB, CH)
    col24 = col23.reshape(NW, NSTAGE, SB, CH)

    h, hp = _tc_proj(x, W_in, b_in.reshape(1, H), dis)
    for _ in range(NUM_LAYERS_):
        u = _sc_agg(hp, row4, col24, zeros2)
        h, hp = _tc_update(h, u[0, :N], u[1, :N], dis, Wp, A_T)
    return h


# speedup vs baseline: 19.4781x; 1.0219x over previous
import functools

import jax
import jax.numpy as jnp
from jax import lax
from jax.experimental import pallas as pl
from jax.experimental.pallas import tpu as pltpu
from jax.experimental.pallas import tpu_sc as plsc

N = 10000
E = 320000
H = 128
HH = H // 2
NUM_LAYERS_ = 4

NC = 2
NS = 16
NW = NC * NS
CH = 80
EPW = E // NW
NCHUNK = EPW // CH
EPS = E // NS
NCHUNK2 = EPS // CH
NP = 10240
RPT = NP // NS

_mesh = plsc.VectorSubcoreMesh(core_axis_name="c", subcore_axis_name="s")


@functools.partial(
    pl.kernel,
    out_type=(
        jax.ShapeDtypeStruct((NW, NCHUNK, CH), jnp.int32),
        jax.ShapeDtypeStruct((NC, NP), jnp.float32),
    ),
    mesh=_mesh,
    scratch_types=[
        pltpu.VMEM((NCHUNK, CH), jnp.int32),
        pltpu.VMEM((NCHUNK, CH), jnp.int32),
        pltpu.VMEM((CH,), jnp.float32),
        pltpu.VMEM((RPT,), jnp.float32),
        pltpu.VMEM_SHARED((NP,), jnp.float32),
    ],
)
def _sc_deg(row3, col3, col2_out, deg_out, row_v, col_v, ones_v, zb_v, acc):
    c = lax.axis_index("c")
    s = lax.axis_index("s")
    w = c * NS + s
    pltpu.sync_copy(row3.at[w], row_v)
    pltpu.sync_copy(col3.at[w], col_v)

    for k in range(CH // 16):
        ones_v[pl.ds(k * 16, 16)] = jnp.full((16,), 1.0, jnp.float32)

    def zfill(i, _):
        zb_v[pl.ds(i * 16, 16)] = jnp.zeros((16,), jnp.float32)
        return 0

    lax.fori_loop(0, RPT // 16, zfill, 0, unroll=8)
    pltpu.sync_copy(zb_v, acc.at[pl.ds(s * RPT, RPT)])

    def rewrite(j, _):
        for k in range(CH // 16):
            r16 = row_v[j, pl.ds(k * 16, 16)]
            c16 = col_v[j, pl.ds(k * 16, 16)]
            col_v[j, pl.ds(k * 16, 16)] = jnp.where(r16 == c16, N, c16)
        return 0

    lax.fori_loop(0, NCHUNK, rewrite, 0)
    pltpu.sync_copy(col_v, col2_out.at[w])
    plsc.subcore_barrier()

    def body(j, _):
        pltpu.sync_copy(ones_v, acc.at[col_v.at[j]], add=True)
        return 0

    lax.fori_loop(0, NCHUNK, body, 0)
    plsc.subcore_barrier()
    pltpu.sync_copy(acc.at[pl.ds(s * RPT, RPT)], deg_out.at[c, pl.ds(s * RPT, RPT)])


SB = 25
NSTAGE = NCHUNK // SB


NR = 3


@functools.partial(
    pl.kernel,
    out_type=jax.ShapeDtypeStruct((NC, NP, H), jnp.float32),
    mesh=_mesh,
    scratch_types=[
        pltpu.VMEM((SB, CH), jnp.int32),
        pltpu.VMEM((SB, CH), jnp.int32),
        pltpu.VMEM((CH, H), jnp.float32),
        pltpu.VMEM((CH, H), jnp.float32),
        pltpu.VMEM((CH, H), jnp.float32),
        pltpu.SemaphoreType.DMA,
        pltpu.SemaphoreType.DMA,
        pltpu.SemaphoreType.DMA,
        pltpu.SemaphoreType.DMA,
        pltpu.SemaphoreType.DMA,
        pltpu.SemaphoreType.DMA,
        pltpu.VMEM_SHARED((NP, H), jnp.float32),
    ],
)
def _sc_agg(hp, row3, col23, u_out, row_v, col_v,
            g0, g1, g2, gs0, gs1, gs2, ss0, ss1, ss2, acc):
    c = lax.axis_index("c")
    s = lax.axis_index("s")
    w = c * NS + s
    gbufs = (g0, g1, g2)
    gsems = (gs0, gs1, gs2)
    ssems = (ss0, ss1, ss2)
    def zrow(i, _):
        for k in range(H // 16):
            g0[i, pl.ds(k * 16, 16)] = jnp.zeros((16,), jnp.float32)
        return 0

    lax.fori_loop(0, CH, zrow, 0, unroll=2)
    for t in range(RPT // CH):
        pltpu.sync_copy(g0, acc.at[pl.ds(s * RPT + t * CH, CH)])
    plsc.subcore_barrier()

    def stage(st, _):
        pltpu.sync_copy(row3.at[w, st], row_v)
        pltpu.sync_copy(col23.at[w, st], col_v)

        for b in range(NR):
            pltpu.async_copy(hp.at[row_v.at[b]], gbufs[b], gsems[b])

        def body(j, _):
            @pl.when(j >= 1)
            def _():
                for b in range(NR):
                    @pl.when((j - 1) % NR == b)
                    def _():
                        pltpu.make_async_copy(
                            gbufs[b], acc.at[col_v.at[j - 1]], ssems[b]).wait()

                        @pl.when(j + NR - 1 < SB)
                        def _():
                            pltpu.async_copy(
                                hp.at[row_v.at[j + NR - 1]], gbufs[b], gsems[b])

            for b in range(NR):
                @pl.when(j % NR == b)
                def _():
                    pltpu.make_async_copy(
                        hp.at[row_v.at[j]], gbufs[b], gsems[b]).wait()
                    pltpu.async_copy(
                        gbufs[b], acc.at[col_v.at[j]], ssems[b], add=True)

            return 0

        lax.fori_loop(0, SB, body, 0)
        for b in range(NR):
            if (SB - 1) % NR == b:
                pltpu.make_async_copy(
                    gbufs[b], acc.at[col_v.at[SB - 1]], ssems[b]).wait()
        return 0

    lax.fori_loop(0, NSTAGE, stage, 0)
    plsc.subcore_barrier()
    pltpu.sync_copy(acc.at[pl.ds(s * RPT, RPT)], u_out.at[c, pl.ds(s * RPT, RPT)])


def _tc_prep_body(deg_ref, wa_ref, wpr_ref, dis_ref, wp_ref, at_ref):
    r = lax.broadcasted_iota(jnp.int32, (H, H), 0)
    col = lax.broadcasted_iota(jnp.int32, (H, H), 1)
    eye = (r == col).astype(jnp.float32)
    w0 = jnp.where(col > r, wpr_ref[:, :H], 0.0)
    w0t = lax.dot_general(w0, eye, (((0,), (0,)), ((), ())),
                          preferred_element_type=jnp.float32)
    w0s = w0 + w0t
    q = wpr_ref[:, H:H + 1]
    rr = wpr_ref[:, H + 1:H + 2]
    d = q * jnp.sum(jnp.abs(w0s), axis=1, keepdims=True) + rr
    wp_ref[...] = w0s + eye * d
    wa = wa_ref[...]
    wat = lax.dot_general(wa, eye, (((0,), (0,)), ((), ())),
                          preferred_element_type=jnp.float32)
    at_ref[...] = wat - wa
    deg = deg_ref[0] + deg_ref[1]
    dis_ref[...] = jnp.where(deg > 0, lax.rsqrt(jnp.maximum(deg, 1e-30)), 0.0)


_tc_prep = pl.pallas_call(
    _tc_prep_body,
    out_shape=(
        jax.ShapeDtypeStruct((NP // 128, 128), jnp.float32),
        jax.ShapeDtypeStruct((H, H), jnp.float32),
        jax.ShapeDtypeStruct((H, H), jnp.float32),
    ),
)

_RB = 2000


def _tc_proj_body(x_ref, wi_ref, b_ref, dis_ref, h_ref, hp_ref):
    hb = lax.dot_general(x_ref[...], wi_ref[...], (((1,), (1,)), ((), ())),
                         preferred_element_type=jnp.float32)
    hb = jnp.maximum(hb + b_ref[...], 0.0)
    h_ref[...] = hb
    hp_ref[...] = dis_ref[...] * hb


_tc_proj = pl.pallas_call(
    _tc_proj_body,
    grid=(N // _RB,),
    in_specs=[
        pl.BlockSpec((_RB, H), lambda i: (i, 0)),
        pl.BlockSpec((H, H), lambda i: (0, 0)),
        pl.BlockSpec((1, H), lambda i: (0, 0)),
        pl.BlockSpec((_RB, 1), lambda i: (i, 0)),
    ],
    out_specs=(
        pl.BlockSpec((_RB, H), lambda i: (i, 0)),
        pl.BlockSpec((_RB, H), lambda i: (i, 0)),
    ),
    out_shape=(
        jax.ShapeDtypeStruct((N, H), jnp.float32),
        jax.ShapeDtypeStruct((N, H), jnp.float32),
    ),
)


def _tc_update_body(h_ref, u0_ref, u1_ref, dis_ref, wp_ref, at_ref,
                    hn_ref, hp_ref):
    usum = u0_ref[...] + u1_ref[...]
    agg = dis_ref[...] * lax.dot_general(
        usum, wp_ref[...], (((1,), (0,)), ((), ())),
        preferred_element_type=jnp.float32)
    t = jnp.maximum(lax.dot_general(
        h_ref[...], at_ref[...], (((1,), (0,)), ((), ())),
        preferred_element_type=jnp.float32), 0.0)
    hn = h_ref[...] + jnp.maximum(jnp.tanh(agg - t), 0.0)
    hn_ref[...] = hn
    hp_ref[...] = dis_ref[...] * hn


_tc_update = pl.pallas_call(
    _tc_update_body,
    grid=(N // _RB,),
    in_specs=[
        pl.BlockSpec((_RB, H), lambda i: (i, 0)),
        pl.BlockSpec((_RB, H), lambda i: (i, 0)),
        pl.BlockSpec((_RB, H), lambda i: (i, 0)),
        pl.BlockSpec((_RB, 1), lambda i: (i, 0)),
        pl.BlockSpec((H, H), lambda i: (0, 0)),
        pl.BlockSpec((H, H), lambda i: (0, 0)),
    ],
    out_specs=(
        pl.BlockSpec((_RB, H), lambda i: (i, 0)),
        pl.BlockSpec((_RB, H), lambda i: (i, 0)),
    ),
    out_shape=(
        jax.ShapeDtypeStruct((N, H), jnp.float32),
        jax.ShapeDtypeStruct((N, H), jnp.float32),
    ),
)


def kernel(x, edge_index, W_in, b_in, W_anti, W_pair):
    row3 = edge_index[0].reshape(NW, NCHUNK, CH)
    col3 = edge_index[1].reshape(NW, NCHUNK, CH)

    col23, deg2 = _sc_deg(row3, col3)
    dis2d, Wp, A_T = _tc_prep(deg2.reshape(NC, NP // 128, 128), W_anti, W_pair)
    dis = dis2d.reshape(NP)[:N].reshape(N, 1)

    row4 = row3.reshape(NW, NSTAGE, SB, CH)
    col24 = col23.reshape(NW, NSTAGE, SB, CH)

    h, hp = _tc_proj(x, W_in, b_in.reshape(1, H), dis)
    for _ in range(NUM_LAYERS_):
        u = _sc_agg(hp, row4, col24)
        h, hp = _tc_update(h, u[0, :N], u[1, :N], dis, Wp, A_T)
    return h


# speedup vs baseline: 19.5603x; 1.0042x over previous
import functools

import jax
import jax.numpy as jnp
from jax import lax
from jax.experimental import pallas as pl
from jax.experimental.pallas import tpu as pltpu
from jax.experimental.pallas import tpu_sc as plsc

N = 10000
E = 320000
H = 128
HH = H // 2
NUM_LAYERS_ = 4

NC = 2
NS = 16
NW = NC * NS
CH = 80
EPW = E // NW
NCHUNK = EPW // CH
EPS = E // NS
NCHUNK2 = EPS // CH
NP = 10240
RPT = NP // NS

_mesh = plsc.VectorSubcoreMesh(core_axis_name="c", subcore_axis_name="s")


@functools.partial(
    pl.kernel,
    out_type=(
        jax.ShapeDtypeStruct((NW, NCHUNK, CH), jnp.int32),
        jax.ShapeDtypeStruct((NC, NP), jnp.float32),
    ),
    mesh=_mesh,
    scratch_types=[
        pltpu.VMEM((NCHUNK, CH), jnp.int32),
        pltpu.VMEM((NCHUNK, CH), jnp.int32),
        pltpu.VMEM((CH,), jnp.float32),
        pltpu.VMEM((RPT,), jnp.float32),
        pltpu.VMEM_SHARED((NP,), jnp.float32),
    ],
)
def _sc_deg(row3, col3, col2_out, deg_out, row_v, col_v, ones_v, zb_v, acc):
    c = lax.axis_index("c")
    s = lax.axis_index("s")
    w = c * NS + s
    pltpu.sync_copy(row3.at[w], row_v)
    pltpu.sync_copy(col3.at[w], col_v)

    for k in range(CH // 16):
        ones_v[pl.ds(k * 16, 16)] = jnp.full((16,), 1.0, jnp.float32)

    def zfill(i, _):
        zb_v[pl.ds(i * 16, 16)] = jnp.zeros((16,), jnp.float32)
        return 0

    lax.fori_loop(0, RPT // 16, zfill, 0, unroll=8)
    pltpu.sync_copy(zb_v, acc.at[pl.ds(s * RPT, RPT)])

    def rewrite(j, _):
        for k in range(CH // 16):
            r16 = row_v[j, pl.ds(k * 16, 16)]
            c16 = col_v[j, pl.ds(k * 16, 16)]
            col_v[j, pl.ds(k * 16, 16)] = jnp.where(r16 == c16, N, c16)
        return 0

    lax.fori_loop(0, NCHUNK, rewrite, 0)
    pltpu.sync_copy(col_v, col2_out.at[w])
    plsc.subcore_barrier()

    def body(j, _):
        pltpu.sync_copy(ones_v, acc.at[col_v.at[j]], add=True)
        return 0

    lax.fori_loop(0, NCHUNK, body, 0)
    plsc.subcore_barrier()
    pltpu.sync_copy(acc.at[pl.ds(s * RPT, RPT)], deg_out.at[c, pl.ds(s * RPT, RPT)])


SB = 25
NSTAGE = NCHUNK // SB


NR = 4


@functools.partial(
    pl.kernel,
    out_type=jax.ShapeDtypeStruct((NC, NP, H), jnp.float32),
    mesh=_mesh,
    scratch_types=[
        pltpu.VMEM((SB, CH), jnp.int32),
        pltpu.VMEM((SB, CH), jnp.int32),
        pltpu.VMEM((CH, H), jnp.float32),
        pltpu.VMEM((CH, H), jnp.float32),
        pltpu.VMEM((CH, H), jnp.float32),
        pltpu.VMEM((CH, H), jnp.float32),
        pltpu.SemaphoreType.DMA,
        pltpu.SemaphoreType.DMA,
        pltpu.SemaphoreType.DMA,
        pltpu.SemaphoreType.DMA,
        pltpu.SemaphoreType.DMA,
        pltpu.SemaphoreType.DMA,
        pltpu.SemaphoreType.DMA,
        pltpu.SemaphoreType.DMA,
        pltpu.VMEM_SHARED((NP, H), jnp.float32),
    ],
)
def _sc_agg(hp, row3, col23, u_out, row_v, col_v,
            g0, g1, g2, g3, gs0, gs1, gs2, gs3, ss0, ss1, ss2, ss3, acc):
    c = lax.axis_index("c")
    s = lax.axis_index("s")
    w = c * NS + s
    gbufs = (g0, g1, g2, g3)
    gsems = (gs0, gs1, gs2, gs3)
    ssems = (ss0, ss1, ss2, ss3)
    def zrow(i, _):
        for k in range(H // 16):
            g0[i, pl.ds(k * 16, 16)] = jnp.zeros((16,), jnp.float32)
        return 0

    lax.fori_loop(0, CH, zrow, 0, unroll=2)
    for t in range(RPT // CH):
        pltpu.sync_copy(g0, acc.at[pl.ds(s * RPT + t * CH, CH)])
    plsc.subcore_barrier()

    def stage(st, _):
        pltpu.sync_copy(row3.at[w, st], row_v)
        pltpu.sync_copy(col23.at[w, st], col_v)

        for b in range(NR):
            pltpu.async_copy(hp.at[row_v.at[b]], gbufs[b], gsems[b])

        def body(j, _):
            @pl.when(j >= 1)
            def _():
                for b in range(NR):
                    @pl.when((j - 1) % NR == b)
                    def _():
                        pltpu.make_async_copy(
                            gbufs[b], acc.at[col_v.at[j - 1]], ssems[b]).wait()

                        @pl.when(j + NR - 1 < SB)
                        def _():
                            pltpu.async_copy(
                                hp.at[row_v.at[j + NR - 1]], gbufs[b], gsems[b])

            for b in range(NR):
                @pl.when(j % NR == b)
                def _():
                    pltpu.make_async_copy(
                        hp.at[row_v.at[j]], gbufs[b], gsems[b]).wait()
                    pltpu.async_copy(
                        gbufs[b], acc.at[col_v.at[j]], ssems[b], add=True)

            return 0

        lax.fori_loop(0, SB, body, 0)
        for b in range(NR):
            if (SB - 1) % NR == b:
                pltpu.make_async_copy(
                    gbufs[b], acc.at[col_v.at[SB - 1]], ssems[b]).wait()
        return 0

    lax.fori_loop(0, NSTAGE, stage, 0)
    plsc.subcore_barrier()
    pltpu.sync_copy(acc.at[pl.ds(s * RPT, RPT)], u_out.at[c, pl.ds(s * RPT, RPT)])


def _tc_prep_body(deg_ref, wa_ref, wpr_ref, dis_ref, wp_ref, at_ref):
    r = lax.broadcasted_iota(jnp.int32, (H, H), 0)
    col = lax.broadcasted_iota(jnp.int32, (H, H), 1)
    eye = (r == col).astype(jnp.float32)
    w0 = jnp.where(col > r, wpr_ref[:, :H], 0.0)
    w0t = lax.dot_general(w0, eye, (((0,), (0,)), ((), ())),
                          preferred_element_type=jnp.float32)
    w0s = w0 + w0t
    q = wpr_ref[:, H:H + 1]
    rr = wpr_ref[:, H + 1:H + 2]
    d = q * jnp.sum(jnp.abs(w0s), axis=1, keepdims=True) + rr
    wp_ref[...] = w0s + eye * d
    wa = wa_ref[...]
    wat = lax.dot_general(wa, eye, (((0,), (0,)), ((), ())),
                          preferred_element_type=jnp.float32)
    at_ref[...] = wat - wa
    deg = deg_ref[0] + deg_ref[1]
    dis_ref[...] = jnp.where(deg > 0, lax.rsqrt(jnp.maximum(deg, 1e-30)), 0.0)


_tc_prep = pl.pallas_call(
    _tc_prep_body,
    out_shape=(
        jax.ShapeDtypeStruct((NP // 128, 128), jnp.float32),
        jax.ShapeDtypeStruct((H, H), jnp.float32),
        jax.ShapeDtypeStruct((H, H), jnp.float32),
    ),
)

_RB = 2000


def _tc_proj_body(x_ref, wi_ref, b_ref, dis_ref, h_ref, hp_ref):
    hb = lax.dot_general(x_ref[...], wi_ref[...], (((1,), (1,)), ((), ())),
                         preferred_element_type=jnp.float32)
    hb = jnp.maximum(hb + b_ref[...], 0.0)
    h_ref[...] = hb
    hp_ref[...] = dis_ref[...] * hb


_tc_proj = pl.pallas_call(
    _tc_proj_body,
    grid=(N // _RB,),
    in_specs=[
        pl.BlockSpec((_RB, H), lambda i: (i, 0)),
        pl.BlockSpec((H, H), lambda i: (0, 0)),
        pl.BlockSpec((1, H), lambda i: (0, 0)),
        pl.BlockSpec((_RB, 1), lambda i: (i, 0)),
    ],
    out_specs=(
        pl.BlockSpec((_RB, H), lambda i: (i, 0)),
        pl.BlockSpec((_RB, H), lambda i: (i, 0)),
    ),
    out_shape=(
        jax.ShapeDtypeStruct((N, H), jnp.float32),
        jax.ShapeDtypeStruct((N, H), jnp.float32),
    ),
)


def _tc_update_body(h_ref, u0_ref, u1_ref, dis_ref, wp_ref, at_ref,
                    hn_ref, hp_ref):
    usum = u0_ref[...] + u1_ref[...]
    agg = dis_ref[...] * lax.dot_general(
        usum, wp_ref[...], (((1,), (0,)), ((), ())),
        preferred_element_type=jnp.float32)
    t = jnp.maximum(lax.dot_general(
        h_ref[...], at_ref[...], (((1,), (0,)), ((), ())),
        preferred_element_type=jnp.float32), 0.0)
    hn = h_ref[...] + jnp.maximum(jnp.tanh(agg - t), 0.0)
    hn_ref[...] = hn
    hp_ref[...] = dis_ref[...] * hn


_tc_update = pl.pallas_call(
    _tc_update_body,
    grid=(N // _RB,),
    in_specs=[
        pl.BlockSpec((_RB, H), lambda i: (i, 0)),
        pl.BlockSpec((_RB, H), lambda i: (i, 0)),
        pl.BlockSpec((_RB, H), lambda i: (i, 0)),
        pl.BlockSpec((_RB, 1), lambda i: (i, 0)),
        pl.BlockSpec((H, H), lambda i: (0, 0)),
        pl.BlockSpec((H, H), lambda i: (0, 0)),
    ],
    out_specs=(
        pl.BlockSpec((_RB, H), lambda i: (i, 0)),
        pl.BlockSpec((_RB, H), lambda i: (i, 0)),
    ),
    out_shape=(
        jax.ShapeDtypeStruct((N, H), jnp.float32),
        jax.ShapeDtypeStruct((N, H), jnp.float32),
    ),
)


def kernel(x, edge_index, W_in, b_in, W_anti, W_pair):
    row3 = edge_index[0].reshape(NW, NCHUNK, CH)
    col3 = edge_index[1].reshape(NW, NCHUNK, CH)

    col23, deg2 = _sc_deg(row3, col3)
    dis2d, Wp, A_T = _tc_prep(deg2.reshape(NC, NP // 128, 128), W_anti, W_pair)
    dis = dis2d.reshape(NP)[:N].reshape(N, 1)

    row4 = row3.reshape(NW, NSTAGE, SB, CH)
    col24 = col23.reshape(NW, NSTAGE, SB, CH)

    h, hp = _tc_proj(x, W_in, b_in.reshape(1, H), dis)
    for _ in range(NUM_LAYERS_):
        u = _sc_agg(hp, row4, col24)
        h, hp = _tc_update(h, u[0, :N], u[1, :N], dis, Wp, A_T)
    return h


# speedup vs baseline: 19.7228x; 1.0083x over previous
import functools

import jax
import jax.numpy as jnp
from jax import lax
from jax.experimental import pallas as pl
from jax.experimental.pallas import tpu as pltpu
from jax.experimental.pallas import tpu_sc as plsc

N = 10000
E = 320000
H = 128
HH = H // 2
NUM_LAYERS_ = 4

NC = 2
NS = 16
NW = NC * NS
CH = 80
EPW = E // NW
NCHUNK = EPW // CH
EPS = E // NS
NCHUNK2 = EPS // CH
NP = 10240
RPT = NP // NS

_mesh = plsc.VectorSubcoreMesh(core_axis_name="c", subcore_axis_name="s")


@functools.partial(
    pl.kernel,
    out_type=(
        jax.ShapeDtypeStruct((NW, NCHUNK, CH), jnp.int32),
        jax.ShapeDtypeStruct((NC, NP), jnp.float32),
    ),
    mesh=_mesh,
    scratch_types=[
        pltpu.VMEM((NCHUNK, CH), jnp.int32),
        pltpu.VMEM((NCHUNK, CH), jnp.int32),
        pltpu.VMEM((CH,), jnp.float32),
        pltpu.VMEM((RPT,), jnp.float32),
        pltpu.VMEM_SHARED((NP,), jnp.float32),
    ],
)
def _sc_deg(row3, col3, col2_out, deg_out, row_v, col_v, ones_v, zb_v, acc):
    c = lax.axis_index("c")
    s = lax.axis_index("s")
    w = c * NS + s
    pltpu.sync_copy(row3.at[w], row_v)
    pltpu.sync_copy(col3.at[w], col_v)

    for k in range(CH // 16):
        ones_v[pl.ds(k * 16, 16)] = jnp.full((16,), 1.0, jnp.float32)

    def zfill(i, _):
        zb_v[pl.ds(i * 16, 16)] = jnp.zeros((16,), jnp.float32)
        return 0

    lax.fori_loop(0, RPT // 16, zfill, 0, unroll=8)
    pltpu.sync_copy(zb_v, acc.at[pl.ds(s * RPT, RPT)])

    def rewrite(j, _):
        for k in range(CH // 16):
            r16 = row_v[j, pl.ds(k * 16, 16)]
            c16 = col_v[j, pl.ds(k * 16, 16)]
            col_v[j, pl.ds(k * 16, 16)] = jnp.where(r16 == c16, N, c16)
        return 0

    lax.fori_loop(0, NCHUNK, rewrite, 0)
    pltpu.sync_copy(col_v, col2_out.at[w])
    plsc.subcore_barrier()

    def body(j, _):
        pltpu.sync_copy(ones_v, acc.at[col_v.at[j]], add=True)
        return 0

    lax.fori_loop(0, NCHUNK, body, 0)
    plsc.subcore_barrier()
    pltpu.sync_copy(acc.at[pl.ds(s * RPT, RPT)], deg_out.at[c, pl.ds(s * RPT, RPT)])


SB = 25
NSTAGE = NCHUNK // SB


NR = 4


@functools.partial(
    pl.kernel,
    out_type=jax.ShapeDtypeStruct((NC, NP, H), jnp.float32),
    mesh=_mesh,
    scratch_types=[
        pltpu.VMEM((SB, CH), jnp.int32),
        pltpu.VMEM((SB, CH), jnp.int32),
        pltpu.VMEM((CH, H), jnp.float32),
        pltpu.VMEM((CH, H), jnp.float32),
        pltpu.VMEM((CH, H), jnp.float32),
        pltpu.VMEM((CH, H), jnp.float32),
        pltpu.SemaphoreType.DMA,
        pltpu.SemaphoreType.DMA,
        pltpu.SemaphoreType.DMA,
        pltpu.SemaphoreType.DMA,
        pltpu.SemaphoreType.DMA,
        pltpu.SemaphoreType.DMA,
        pltpu.SemaphoreType.DMA,
        pltpu.SemaphoreType.DMA,
        pltpu.VMEM_SHARED((NP, H), jnp.float32),
    ],
)
def _sc_agg(hp, row3, col23, u_out, row_v, col_v,
            g0, g1, g2, g3, gs0, gs1, gs2, gs3, ss0, ss1, ss2, ss3, acc):
    c = lax.axis_index("c")
    s = lax.axis_index("s")
    w = c * NS + s
    gbufs = (g0, g1, g2, g3)
    gsems = (gs0, gs1, gs2, gs3)
    ssems = (ss0, ss1, ss2, ss3)
    def zrow(i, _):
        for k in range(H // 16):
            g0[i, pl.ds(k * 16, 16)] = jnp.zeros((16,), jnp.float32)
        return 0

    lax.fori_loop(0, CH, zrow, 0, unroll=2)
    for t in range(RPT // CH):
        pltpu.sync_copy(g0, acc.at[pl.ds(s * RPT + t * CH, CH)])
    plsc.subcore_barrier()

    def stage(st, _):
        pltpu.sync_copy(row3.at[w, st], row_v)
        pltpu.sync_copy(col23.at[w, st], col_v)

        for b in range(NR):
            pltpu.async_copy(hp.at[row_v.at[b]], gbufs[b], gsems[b])

        def body(j, _):
            @pl.when(j >= 1)
            def _():
                for b in range(NR):
                    @pl.when((j - 1) % NR == b)
                    def _():
                        pltpu.make_async_copy(
                            gbufs[b], acc.at[col_v.at[j - 1]], ssems[b]).wait()

                        @pl.when(j + NR - 1 < SB)
                        def _():
                            pltpu.async_copy(
                                hp.at[row_v.at[j + NR - 1]], gbufs[b], gsems[b])

            for b in range(NR):
                @pl.when(j % NR == b)
                def _():
                    pltpu.make_async_copy(
                        hp.at[row_v.at[j]], gbufs[b], gsems[b]).wait()
                    pltpu.async_copy(
                        gbufs[b], acc.at[col_v.at[j]], ssems[b], add=True)

            return 0

        lax.fori_loop(0, SB, body, 0)
        for b in range(NR):
            if (SB - 1) % NR == b:
                pltpu.make_async_copy(
                    gbufs[b], acc.at[col_v.at[SB - 1]], ssems[b]).wait()
        return 0

    lax.fori_loop(0, NSTAGE, stage, 0)
    plsc.subcore_barrier()
    pltpu.sync_copy(acc.at[pl.ds(s * RPT, RPT)], u_out.at[c, pl.ds(s * RPT, RPT)])


_RB = 2000


def _tc_proj_body(x_ref, wi_ref, b_ref, deg_ref, wa_ref, wpr_ref,
                  h_ref, hp_ref, dis_ref, wp_ref, at_ref):
    i = pl.program_id(0)

    @pl.when(i == 0)
    def _():
        r = lax.broadcasted_iota(jnp.int32, (H, H), 0)
        col = lax.broadcasted_iota(jnp.int32, (H, H), 1)
        eye = (r == col).astype(jnp.float32)
        w0 = jnp.where(col > r, wpr_ref[:, :H], 0.0)
        w0t = lax.dot_general(w0, eye, (((0,), (0,)), ((), ())),
                              preferred_element_type=jnp.float32)
        w0s = w0 + w0t
        q = wpr_ref[:, H:H + 1]
        rr = wpr_ref[:, H + 1:H + 2]
        d = q * jnp.sum(jnp.abs(w0s), axis=1, keepdims=True) + rr
        wp_ref[...] = w0s + eye * d
        wa = wa_ref[...]
        wat = lax.dot_general(wa, eye, (((0,), (0,)), ((), ())),
                              preferred_element_type=jnp.float32)
        at_ref[...] = wat - wa

    deg = deg_ref[...]
    dis = jnp.where(deg > 0, lax.rsqrt(jnp.maximum(deg, 1e-30)), 0.0)
    dis_ref[...] = dis
    hb = lax.dot_general(x_ref[...], wi_ref[...], (((1,), (1,)), ((), ())),
                         preferred_element_type=jnp.float32)
    hb = jnp.maximum(hb + b_ref[...], 0.0)
    h_ref[...] = hb
    hp_ref[...] = dis * hb


_tc_proj = pl.pallas_call(
    _tc_proj_body,
    grid=(N // _RB,),
    in_specs=[
        pl.BlockSpec((_RB, H), lambda i: (i, 0)),
        pl.BlockSpec((H, H), lambda i: (0, 0)),
        pl.BlockSpec((1, H), lambda i: (0, 0)),
        pl.BlockSpec((_RB, 1), lambda i: (i, 0)),
        pl.BlockSpec((H, H), lambda i: (0, 0)),
        pl.BlockSpec((H, H + 2), lambda i: (0, 0)),
    ],
    out_specs=(
        pl.BlockSpec((_RB, H), lambda i: (i, 0)),
        pl.BlockSpec((_RB, H), lambda i: (i, 0)),
        pl.BlockSpec((_RB, 1), lambda i: (i, 0)),
        pl.BlockSpec((H, H), lambda i: (0, 0)),
        pl.BlockSpec((H, H), lambda i: (0, 0)),
    ),
    out_shape=(
        jax.ShapeDtypeStruct((N, H), jnp.float32),
        jax.ShapeDtypeStruct((N, H), jnp.float32),
        jax.ShapeDtypeStruct((N, 1), jnp.float32),
        jax.ShapeDtypeStruct((H, H), jnp.float32),
        jax.ShapeDtypeStruct((H, H), jnp.float32),
    ),
)


def _tc_update_body(h_ref, u0_ref, u1_ref, dis_ref, wp_ref, at_ref,
                    hn_ref, hp_ref):
    usum = u0_ref[...] + u1_ref[...]
    agg = dis_ref[...] * lax.dot_general(
        usum, wp_ref[...], (((1,), (0,)), ((), ())),
        preferred_element_type=jnp.float32)
    t = jnp.maximum(lax.dot_general(
        h_ref[...], at_ref[...], (((1,), (0,)), ((), ())),
        preferred_element_type=jnp.float32), 0.0)
    hn = h_ref[...] + jnp.maximum(jnp.tanh(agg - t), 0.0)
    hn_ref[...] = hn
    hp_ref[...] = dis_ref[...] * hn


_tc_update = pl.pallas_call(
    _tc_update_body,
    grid=(N // _RB,),
    in_specs=[
        pl.BlockSpec((_RB, H), lambda i: (i, 0)),
        pl.BlockSpec((_RB, H), lambda i: (i, 0)),
        pl.BlockSpec((_RB, H), lambda i: (i, 0)),
        pl.BlockSpec((_RB, 1), lambda i: (i, 0)),
        pl.BlockSpec((H, H), lambda i: (0, 0)),
        pl.BlockSpec((H, H), lambda i: (0, 0)),
    ],
    out_specs=(
        pl.BlockSpec((_RB, H), lambda i: (i, 0)),
        pl.BlockSpec((_RB, H), lambda i: (i, 0)),
    ),
    out_shape=(
        jax.ShapeDtypeStruct((N, H), jnp.float32),
        jax.ShapeDtypeStruct((N, H), jnp.float32),
    ),
)


def kernel(x, edge_index, W_in, b_in, W_anti, W_pair):
    row3 = edge_index[0].reshape(NW, NCHUNK, CH)
    col3 = edge_index[1].reshape(NW, NCHUNK, CH)

    col23, deg2 = _sc_deg(row3, col3)
    deg_col = (deg2[0] + deg2[1])[:N].reshape(N, 1)

    row4 = row3.reshape(NW, NSTAGE, SB, CH)
    col24 = col23.reshape(NW, NSTAGE, SB, CH)

    h, hp, dis, Wp, A_T = _tc_proj(x, W_in, b_in.reshape(1, H), deg_col,
                                   W_anti, W_pair)
    for _ in range(NUM_LAYERS_):
        u = _sc_agg(hp, row4, col24)
        h, hp = _tc_update(h, u[0, :N], u[1, :N], dis, Wp, A_T)
    return h


# speedup vs baseline: 19.9206x; 1.0100x over previous
import functools

import jax
import jax.numpy as jnp
from jax import lax
from jax.experimental import pallas as pl
from jax.experimental.pallas import tpu as pltpu
from jax.experimental.pallas import tpu_sc as plsc

N = 10000
E = 320000
H = 128
HH = H // 2
NUM_LAYERS_ = 4

NC = 2
NS = 16
NW = NC * NS
CH = 80
EPW = E // NW
NCHUNK = EPW // CH
EPS = E // NS
NCHUNK2 = EPS // CH
NP = 10240
RPT = NP // NS

_mesh = plsc.VectorSubcoreMesh(core_axis_name="c", subcore_axis_name="s")


@functools.partial(
    pl.kernel,
    out_type=(
        jax.ShapeDtypeStruct((NW, NCHUNK, CH), jnp.int32),
        jax.ShapeDtypeStruct((NC, NP), jnp.float32),
    ),
    mesh=_mesh,
    scratch_types=[
        pltpu.VMEM((NCHUNK, CH), jnp.int32),
        pltpu.VMEM((NCHUNK, CH), jnp.int32),
        pltpu.VMEM((CH,), jnp.float32),
        pltpu.VMEM((RPT,), jnp.float32),
        pltpu.VMEM_SHARED((NP,), jnp.float32),
    ],
)
def _sc_deg(row3, col3, col2_out, deg_out, row_v, col_v, ones_v, zb_v, acc):
    c = lax.axis_index("c")
    s = lax.axis_index("s")
    w = c * NS + s
    pltpu.sync_copy(row3.at[w], row_v)
    pltpu.sync_copy(col3.at[w], col_v)

    for k in range(CH // 16):
        ones_v[pl.ds(k * 16, 16)] = jnp.full((16,), 1.0, jnp.float32)

    def zfill(i, _):
        zb_v[pl.ds(i * 16, 16)] = jnp.zeros((16,), jnp.float32)
        return 0

    lax.fori_loop(0, RPT // 16, zfill, 0, unroll=8)
    pltpu.sync_copy(zb_v, acc.at[pl.ds(s * RPT, RPT)])

    def rewrite(j, _):
        for k in range(CH // 16):
            r16 = row_v[j, pl.ds(k * 16, 16)]
            c16 = col_v[j, pl.ds(k * 16, 16)]
            col_v[j, pl.ds(k * 16, 16)] = jnp.where(r16 == c16, N, c16)
        return 0

    lax.fori_loop(0, NCHUNK, rewrite, 0)
    pltpu.sync_copy(col_v, col2_out.at[w])
    plsc.subcore_barrier()

    def body(j, _):
        pltpu.sync_copy(ones_v, acc.at[col_v.at[j]], add=True)
        return 0

    lax.fori_loop(0, NCHUNK, body, 0)
    plsc.subcore_barrier()
    pltpu.sync_copy(acc.at[pl.ds(s * RPT, RPT)], deg_out.at[c, pl.ds(s * RPT, RPT)])


SB = 25
NSTAGE = NCHUNK // SB


NR = 4


@functools.partial(
    pl.kernel,
    out_type=jax.ShapeDtypeStruct((NC, NP, H), jnp.float32),
    mesh=_mesh,
    scratch_types=[
        pltpu.VMEM((SB, CH), jnp.int32),
        pltpu.VMEM((SB, CH), jnp.int32),
        pltpu.VMEM((CH, H), jnp.float32),
        pltpu.VMEM((CH, H), jnp.float32),
        pltpu.VMEM((CH, H), jnp.float32),
        pltpu.VMEM((CH, H), jnp.float32),
        pltpu.SemaphoreType.DMA,
        pltpu.SemaphoreType.DMA,
        pltpu.SemaphoreType.DMA,
        pltpu.SemaphoreType.DMA,
        pltpu.SemaphoreType.DMA,
        pltpu.SemaphoreType.DMA,
        pltpu.SemaphoreType.DMA,
        pltpu.SemaphoreType.DMA,
        pltpu.VMEM_SHARED((NP, H), jnp.float32),
    ],
)
def _sc_agg(hp, row3, col23, u_out, row_v, col_v,
            g0, g1, g2, g3, gs0, gs1, gs2, gs3, ss0, ss1, ss2, ss3, acc):
    c = lax.axis_index("c")
    s = lax.axis_index("s")
    w = c * NS + s
    gbufs = (g0, g1, g2, g3)
    gsems = (gs0, gs1, gs2, gs3)
    ssems = (ss0, ss1, ss2, ss3)
    pltpu.sync_copy(row3.at[w, 0], row_v)
    pltpu.sync_copy(col23.at[w, 0], col_v)
    for b in range(1, NR):
        pltpu.async_copy(hp.at[row_v.at[b]], gbufs[b], gsems[b])

    def zrow(i, _):
        for k in range(H // 16):
            g0[i, pl.ds(k * 16, 16)] = jnp.zeros((16,), jnp.float32)
        return 0

    lax.fori_loop(0, CH, zrow, 0, unroll=2)
    for t in range(RPT // CH):
        pltpu.sync_copy(g0, acc.at[pl.ds(s * RPT + t * CH, CH)])
    pltpu.async_copy(hp.at[row_v.at[0]], g0, gs0)
    plsc.subcore_barrier()

    def stage(st, _):
        @pl.when(st > 0)
        def _():
            pltpu.sync_copy(row3.at[w, st], row_v)
            pltpu.sync_copy(col23.at[w, st], col_v)

            for b in range(NR):
                pltpu.async_copy(hp.at[row_v.at[b]], gbufs[b], gsems[b])

        def body(j, _):
            @pl.when(j >= 1)
            def _():
                for b in range(NR):
                    @pl.when((j - 1) % NR == b)
                    def _():
                        pltpu.make_async_copy(
                            gbufs[b], acc.at[col_v.at[j - 1]], ssems[b]).wait()

                        @pl.when(j + NR - 1 < SB)
                        def _():
                            pltpu.async_copy(
                                hp.at[row_v.at[j + NR - 1]], gbufs[b], gsems[b])

            for b in range(NR):
                @pl.when(j % NR == b)
                def _():
                    pltpu.make_async_copy(
                        hp.at[row_v.at[j]], gbufs[b], gsems[b]).wait()
                    pltpu.async_copy(
                        gbufs[b], acc.at[col_v.at[j]], ssems[b], add=True)

            return 0

        lax.fori_loop(0, SB, body, 0)
        for b in range(NR):
            if (SB - 1) % NR == b:
                pltpu.make_async_copy(
                    gbufs[b], acc.at[col_v.at[SB - 1]], ssems[b]).wait()
        return 0

    lax.fori_loop(0, NSTAGE, stage, 0)
    plsc.subcore_barrier()
    pltpu.sync_copy(acc.at[pl.ds(s * RPT, RPT)], u_out.at[c, pl.ds(s * RPT, RPT)])


_RB = 2000


def _tc_proj_body(x_ref, wi_ref, b_ref, deg_ref, wa_ref, wpr_ref,
                  h_ref, hp_ref, dis_ref, wp_ref, at_ref):
    i = pl.program_id(0)

    @pl.when(i == 0)
    def _():
        r = lax.broadcasted_iota(jnp.int32, (H, H), 0)
        col = lax.broadcasted_iota(jnp.int32, (H, H), 1)
        eye = (r == col).astype(jnp.float32)
        w0 = jnp.where(col > r, wpr_ref[:, :H], 0.0)
        w0t = lax.dot_general(w0, eye, (((0,), (0,)), ((), ())),
                              preferred_element_type=jnp.float32)
        w0s = w0 + w0t
        q = wpr_ref[:, H:H + 1]
        rr = wpr_ref[:, H + 1:H + 2]
        d = q * jnp.sum(jnp.abs(w0s), axis=1, keepdims=True) + rr
        wp_ref[...] = w0s + eye * d
        wa = wa_ref[...]
        wat = lax.dot_general(wa, eye, (((0,), (0,)), ((), ())),
                              preferred_element_type=jnp.float32)
        at_ref[...] = wat - wa

    deg = deg_ref[...]
    dis = jnp.where(deg > 0, lax.rsqrt(jnp.maximum(deg, 1e-30)), 0.0)
    dis_ref[...] = dis
    hb = lax.dot_general(x_ref[...], wi_ref[...], (((1,), (1,)), ((), ())),
                         preferred_element_type=jnp.float32)
    hb = jnp.maximum(hb + b_ref[...], 0.0)
    h_ref[...] = hb
    hp_ref[...] = dis * hb


_tc_proj = pl.pallas_call(
    _tc_proj_body,
    grid=(N // _RB,),
    in_specs=[
        pl.BlockSpec((_RB, H), lambda i: (i, 0)),
        pl.BlockSpec((H, H), lambda i: (0, 0)),
        pl.BlockSpec((1, H), lambda i: (0, 0)),
        pl.BlockSpec((_RB, 1), lambda i: (i, 0)),
        pl.BlockSpec((H, H), lambda i: (0, 0)),
        pl.BlockSpec((H, H + 2), lambda i: (0, 0)),
    ],
    out_specs=(
        pl.BlockSpec((_RB, H), lambda i: (i, 0)),
        pl.BlockSpec((_RB, H), lambda i: (i, 0)),
        pl.BlockSpec((_RB, 1), lambda i: (i, 0)),
        pl.BlockSpec((H, H), lambda i: (0, 0)),
        pl.BlockSpec((H, H), lambda i: (0, 0)),
    ),
    out_shape=(
        jax.ShapeDtypeStruct((N, H), jnp.float32),
        jax.ShapeDtypeStruct((N, H), jnp.float32),
        jax.ShapeDtypeStruct((N, 1), jnp.float32),
        jax.ShapeDtypeStruct((H, H), jnp.float32),
        jax.ShapeDtypeStruct((H, H), jnp.float32),
    ),
)


def _tc_update_body(h_ref, u0_ref, u1_ref, dis_ref, wp_ref, at_ref,
                    hn_ref, hp_ref):
    usum = u0_ref[...] + u1_ref[...]
    agg = dis_ref[...] * lax.dot_general(
        usum, wp_ref[...], (((1,), (0,)), ((), ())),
        preferred_element_type=jnp.float32)
    t = jnp.maximum(lax.dot_general(
        h_ref[...], at_ref[...], (((1,), (0,)), ((), ())),
        preferred_element_type=jnp.float32), 0.0)
    hn = h_ref[...] + jnp.maximum(jnp.tanh(agg - t), 0.0)
    hn_ref[...] = hn
    hp_ref[...] = dis_ref[...] * hn


_tc_update = pl.pallas_call(
    _tc_update_body,
    grid=(N // _RB,),
    in_specs=[
        pl.BlockSpec((_RB, H), lambda i: (i, 0)),
        pl.BlockSpec((_RB, H), lambda i: (i, 0)),
        pl.BlockSpec((_RB, H), lambda i: (i, 0)),
        pl.BlockSpec((_RB, 1), lambda i: (i, 0)),
        pl.BlockSpec((H, H), lambda i: (0, 0)),
        pl.BlockSpec((H, H), lambda i: (0, 0)),
    ],
    out_specs=(
        pl.BlockSpec((_RB, H), lambda i: (i, 0)),
        pl.BlockSpec((_RB, H), lambda i: (i, 0)),
    ),
    out_shape=(
        jax.ShapeDtypeStruct((N, H), jnp.float32),
        jax.ShapeDtypeStruct((N, H), jnp.float32),
    ),
)


def kernel(x, edge_index, W_in, b_in, W_anti, W_pair):
    row3 = edge_index[0].reshape(NW, NCHUNK, CH)
    col3 = edge_index[1].reshape(NW, NCHUNK, CH)

    col23, deg2 = _sc_deg(row3, col3)
    deg_col = (deg2[0] + deg2[1])[:N].reshape(N, 1)

    row4 = row3.reshape(NW, NSTAGE, SB, CH)
    col24 = col23.reshape(NW, NSTAGE, SB, CH)

    h, hp, dis, Wp, A_T = _tc_proj(x, W_in, b_in.reshape(1, H), deg_col,
                                   W_anti, W_pair)
    for _ in range(NUM_LAYERS_):
        u = _sc_agg(hp, row4, col24)
        h, hp = _tc_update(h, u[0, :N], u[1, :N], dis, Wp, A_T)
    return h


# speedup vs baseline: 20.2578x; 1.0169x over previous
import functools

import jax
import jax.numpy as jnp
from jax import lax
from jax.experimental import pallas as pl
from jax.experimental.pallas import tpu as pltpu
from jax.experimental.pallas import tpu_sc as plsc

N = 10000
E = 320000
H = 128
HH = H // 2
NUM_LAYERS_ = 4

NC = 2
NS = 16
NW = NC * NS
CH = 80
EPW = E // NW
NCHUNK = EPW // CH
EPS = E // NS
NCHUNK2 = EPS // CH
NP = 10240
RPT = NP // NS

_mesh = plsc.VectorSubcoreMesh(core_axis_name="c", subcore_axis_name="s")


@functools.partial(
    pl.kernel,
    out_type=(
        jax.ShapeDtypeStruct((NW, NCHUNK, CH), jnp.int32),
        jax.ShapeDtypeStruct((NC, NP), jnp.float32),
    ),
    mesh=_mesh,
    scratch_types=[
        pltpu.VMEM((NCHUNK, CH), jnp.int32),
        pltpu.VMEM((NCHUNK, CH), jnp.int32),
        pltpu.VMEM((CH,), jnp.float32),
        pltpu.VMEM((RPT,), jnp.float32),
        pltpu.VMEM_SHARED((NP,), jnp.float32),
    ],
)
def _sc_deg(row3, col3, col2_out, deg_out, row_v, col_v, ones_v, zb_v, acc):
    c = lax.axis_index("c")
    s = lax.axis_index("s")
    w = c * NS + s
    pltpu.sync_copy(row3.at[w], row_v)
    pltpu.sync_copy(col3.at[w], col_v)

    for k in range(CH // 16):
        ones_v[pl.ds(k * 16, 16)] = jnp.full((16,), 1.0, jnp.float32)

    def zfill(i, _):
        zb_v[pl.ds(i * 16, 16)] = jnp.zeros((16,), jnp.float32)
        return 0

    lax.fori_loop(0, RPT // 16, zfill, 0, unroll=8)
    pltpu.sync_copy(zb_v, acc.at[pl.ds(s * RPT, RPT)])

    def rewrite(j, _):
        for k in range(CH // 16):
            r16 = row_v[j, pl.ds(k * 16, 16)]
            c16 = col_v[j, pl.ds(k * 16, 16)]
            col_v[j, pl.ds(k * 16, 16)] = jnp.where(r16 == c16, N, c16)
        return 0

    lax.fori_loop(0, NCHUNK, rewrite, 0)
    pltpu.sync_copy(col_v, col2_out.at[w])
    plsc.subcore_barrier()

    def body(j, _):
        pltpu.sync_copy(ones_v, acc.at[col_v.at[j]], add=True)
        return 0

    lax.fori_loop(0, NCHUNK, body, 0)
    plsc.subcore_barrier()
    pltpu.sync_copy(acc.at[pl.ds(s * RPT, RPT)], deg_out.at[c, pl.ds(s * RPT, RPT)])


SB = 25
NSTAGE = NCHUNK // SB


NR = 4


@functools.partial(
    pl.kernel,
    out_type=jax.ShapeDtypeStruct((NC, NP, H), jnp.float32),
    mesh=_mesh,
    scratch_types=[
        pltpu.VMEM((SB, CH), jnp.int32),
        pltpu.VMEM((SB, CH), jnp.int32),
        pltpu.VMEM((CH, H), jnp.float32),
        pltpu.VMEM((CH, H), jnp.float32),
        pltpu.VMEM((CH, H), jnp.float32),
        pltpu.VMEM((CH, H), jnp.float32),
        pltpu.SemaphoreType.DMA,
        pltpu.SemaphoreType.DMA,
        pltpu.SemaphoreType.DMA,
        pltpu.SemaphoreType.DMA,
        pltpu.SemaphoreType.DMA,
        pltpu.SemaphoreType.DMA,
        pltpu.SemaphoreType.DMA,
        pltpu.SemaphoreType.DMA,
        pltpu.VMEM_SHARED((NP, H), jnp.float32),
    ],
)
def _sc_agg(hp, row3, col23, u_out, row_v, col_v,
            g0, g1, g2, g3, gs0, gs1, gs2, gs3, ss0, ss1, ss2, ss3, acc):
    c = lax.axis_index("c")
    s = lax.axis_index("s")
    w = c * NS + s
    gbufs = (g0, g1, g2, g3)
    gsems = (gs0, gs1, gs2, gs3)
    ssems = (ss0, ss1, ss2, ss3)
    pltpu.sync_copy(row3.at[w, 0], row_v)
    pltpu.sync_copy(col23.at[w, 0], col_v)
    for b in range(1, NR):
        pltpu.async_copy(hp.at[row_v.at[b]], gbufs[b], gsems[b])

    def zrow(i, _):
        for k in range(H // 16):
            g0[i, pl.ds(k * 16, 16)] = jnp.zeros((16,), jnp.float32)
        return 0

    lax.fori_loop(0, CH, zrow, 0, unroll=2)
    for t in range(RPT // CH):
        pltpu.sync_copy(g0, acc.at[pl.ds(s * RPT + t * CH, CH)])
    pltpu.async_copy(hp.at[row_v.at[0]], g0, gs0)
    plsc.subcore_barrier()

    def stage(st, _):
        @pl.when(st > 0)
        def _():
            pltpu.sync_copy(row3.at[w, st], row_v)
            pltpu.sync_copy(col23.at[w, st], col_v)

            for b in range(NR):
                pltpu.async_copy(hp.at[row_v.at[b]], gbufs[b], gsems[b])

        def body(j, _):
            @pl.when(j >= 1)
            def _():
                for b in range(NR):
                    @pl.when((j - 1) % NR == b)
                    def _():
                        pltpu.make_async_copy(
                            gbufs[b], acc.at[col_v.at[j - 1]], ssems[b]).wait()

                        @pl.when(j + NR - 1 < SB)
                        def _():
                            pltpu.async_copy(
                                hp.at[row_v.at[j + NR - 1]], gbufs[b], gsems[b])

            for b in range(NR):
                @pl.when(j % NR == b)
                def _():
                    pltpu.make_async_copy(
                        hp.at[row_v.at[j]], gbufs[b], gsems[b]).wait()
                    pltpu.async_copy(
                        gbufs[b], acc.at[col_v.at[j]], ssems[b], add=True)

            return 0

        lax.fori_loop(0, SB, body, 0)
        for b in range(NR):
            if (SB - 1) % NR == b:
                pltpu.make_async_copy(
                    gbufs[b], acc.at[col_v.at[SB - 1]], ssems[b]).wait()
        return 0

    lax.fori_loop(0, NSTAGE, stage, 0)
    plsc.subcore_barrier()
    pltpu.sync_copy(acc.at[pl.ds(s * RPT, RPT)], u_out.at[c, pl.ds(s * RPT, RPT)])


_RB = 5000


def _tc_proj_body(x_ref, wi_ref, b_ref, deg_ref, wa_ref, wpr_ref,
                  h_ref, hp_ref, dis_ref, wp_ref, at_ref):
    i = pl.program_id(0)

    @pl.when(i == 0)
    def _():
        r = lax.broadcasted_iota(jnp.int32, (H, H), 0)
        col = lax.broadcasted_iota(jnp.int32, (H, H), 1)
        eye = (r == col).astype(jnp.float32)
        w0 = jnp.where(col > r, wpr_ref[:, :H], 0.0)
        w0t = lax.dot_general(w0, eye, (((0,), (0,)), ((), ())),
                              preferred_element_type=jnp.float32)
        w0s = w0 + w0t
        q = wpr_ref[:, H:H + 1]
        rr = wpr_ref[:, H + 1:H + 2]
        d = q * jnp.sum(jnp.abs(w0s), axis=1, keepdims=True) + rr
        wp_ref[...] = w0s + eye * d
        wa = wa_ref[...]
        wat = lax.dot_general(wa, eye, (((0,), (0,)), ((), ())),
                              preferred_element_type=jnp.float32)
        at_ref[...] = wat - wa

    deg = deg_ref[...]
    dis = jnp.where(deg > 0, lax.rsqrt(jnp.maximum(deg, 1e-30)), 0.0)
    dis_ref[...] = dis
    hb = lax.dot_general(x_ref[...], wi_ref[...], (((1,), (1,)), ((), ())),
                         preferred_element_type=jnp.float32)
    hb = jnp.maximum(hb + b_ref[...], 0.0)
    h_ref[...] = hb
    hp_ref[...] = dis * hb


_tc_proj = pl.pallas_call(
    _tc_proj_body,
    grid=(N // _RB,),
    in_specs=[
        pl.BlockSpec((_RB, H), lambda i: (i, 0)),
        pl.BlockSpec((H, H), lambda i: (0, 0)),
        pl.BlockSpec((1, H), lambda i: (0, 0)),
        pl.BlockSpec((_RB, 1), lambda i: (i, 0)),
        pl.BlockSpec((H, H), lambda i: (0, 0)),
        pl.BlockSpec((H, H + 2), lambda i: (0, 0)),
    ],
    out_specs=(
        pl.BlockSpec((_RB, H), lambda i: (i, 0)),
        pl.BlockSpec((_RB, H), lambda i: (i, 0)),
        pl.BlockSpec((_RB, 1), lambda i: (i, 0)),
        pl.BlockSpec((H, H), lambda i: (0, 0)),
        pl.BlockSpec((H, H), lambda i: (0, 0)),
    ),
    out_shape=(
        jax.ShapeDtypeStruct((N, H), jnp.float32),
        jax.ShapeDtypeStruct((N, H), jnp.float32),
        jax.ShapeDtypeStruct((N, 1), jnp.float32),
        jax.ShapeDtypeStruct((H, H), jnp.float32),
        jax.ShapeDtypeStruct((H, H), jnp.float32),
    ),
)


def _tc_update_body(h_ref, u0_ref, u1_ref, dis_ref, wp_ref, at_ref,
                    hn_ref, hp_ref):
    usum = u0_ref[...] + u1_ref[...]
    agg = dis_ref[...] * lax.dot_general(
        usum, wp_ref[...], (((1,), (0,)), ((), ())),
        preferred_element_type=jnp.float32)
    t = jnp.maximum(lax.dot_general(
        h_ref[...], at_ref[...], (((1,), (0,)), ((), ())),
        preferred_element_type=jnp.float32), 0.0)
    hn = h_ref[...] + jnp.maximum(jnp.tanh(agg - t), 0.0)
    hn_ref[...] = hn
    hp_ref[...] = dis_ref[...] * hn


_tc_update = pl.pallas_call(
    _tc_update_body,
    grid=(N // _RB,),
    in_specs=[
        pl.BlockSpec((_RB, H), lambda i: (i, 0)),
        pl.BlockSpec((_RB, H), lambda i: (i, 0)),
        pl.BlockSpec((_RB, H), lambda i: (i, 0)),
        pl.BlockSpec((_RB, 1), lambda i: (i, 0)),
        pl.BlockSpec((H, H), lambda i: (0, 0)),
        pl.BlockSpec((H, H), lambda i: (0, 0)),
    ],
    out_specs=(
        pl.BlockSpec((_RB, H), lambda i: (i, 0)),
        pl.BlockSpec((_RB, H), lambda i: (i, 0)),
    ),
    out_shape=(
        jax.ShapeDtypeStruct((N, H), jnp.float32),
        jax.ShapeDtypeStruct((N, H), jnp.float32),
    ),
)


def kernel(x, edge_index, W_in, b_in, W_anti, W_pair):
    row3 = edge_index[0].reshape(NW, NCHUNK, CH)
    col3 = edge_index[1].reshape(NW, NCHUNK, CH)

    col23, deg2 = _sc_deg(row3, col3)
    deg_col = (deg2[0] + deg2[1])[:N].reshape(N, 1)

    row4 = row3.reshape(NW, NSTAGE, SB, CH)
    col24 = col23.reshape(NW, NSTAGE, SB, CH)

    h, hp, dis, Wp, A_T = _tc_proj(x, W_in, b_in.reshape(1, H), deg_col,
                                   W_anti, W_pair)
    for _ in range(NUM_LAYERS_):
        u = _sc_agg(hp, row4, col24)
        h, hp = _tc_update(h, u[0, :N], u[1, :N], dis, Wp, A_T)
    return h
